# Initial kernel scaffold; baseline (speedup 1.0000x reference)
import jax
import jax.numpy as jnp
from jax.experimental import pallas as pl


def kernel(x, y, edge_index, edge_weight, t, params):
    raise NotImplementedError("write your pallas kernel here")


# speedup vs baseline: 4.4664x; 4.4664x over previous
import jax
import jax.numpy as jnp
import numpy as np
from jax.experimental import pallas as pl

_N = 10000
_E = 320000
_D = 128
_HID = 256
_OUT = 128
_H = 8
_C = 32
_ED = 16

_TILE_E = 2000
_TILE_N = 1000

_HIGH = jax.lax.Precision.HIGHEST


def _pad_rows(a, t):
    r = a.shape[0]
    p = (-r) % t
    return jnp.pad(a, ((0, p), (0, 0))) if p else a


def _row_pallas(body, out_dim, tile, tiled, full=()):
    r0 = tiled[0].shape[0]
    tp = [_pad_rows(a, tile) for a in tiled]
    rr = tp[0].shape[0]
    grid = rr // tile
    in_specs = [pl.BlockSpec((tile, a.shape[1]), lambda i: (i, 0)) for a in tp]
    in_specs += [
        pl.BlockSpec(a.shape, lambda i, n=a.ndim: (0,) * n) for a in full
    ]
    out = pl.pallas_call(
        body,
        grid=(grid,),
        in_specs=in_specs,
        out_specs=pl.BlockSpec((tile, out_dim), lambda i: (i, 0)),
        out_shape=jax.ShapeDtypeStruct((rr, out_dim), jnp.float32),
    )(*tp, *full)
    return out[:r0] if rr != r0 else out


def _bn_body(x_ref, g_ref, b_ref, o_ref):
    x = x_ref[...]
    m = jnp.mean(x, axis=0, keepdims=True)
    v = jnp.mean((x - m) ** 2, axis=0, keepdims=True)
    o_ref[...] = (x - m) * (1.0 / jnp.sqrt(v + 1e-5)) * g_ref[...] + b_ref[...]


def _bn(x, g, b):
    return pl.pallas_call(
        _bn_body,
        out_shape=jax.ShapeDtypeStruct(x.shape, jnp.float32),
    )(x, g.reshape(1, -1), b.reshape(1, -1))


def _scale_body(x_ref, w_ref, o_ref):
    o_ref[...] = x_ref[...] * w_ref[...]


def _escore_body(xls_ref, xrd_ref, ee_ref, a_ref, o_ref):
    m = xls_ref[...] + xrd_ref[...] + ee_ref[...]
    m = jnp.where(m >= 0, m, 0.2 * m)
    o_ref[...] = jnp.dot(
        m, a_ref[...], preferred_element_type=jnp.float32, precision=_HIGH
    )


def _exp_body(a_ref, mx_ref, o_ref):
    o_ref[...] = jnp.exp(a_ref[...] - mx_ref[...])


def _wout_body(xls_ref, a_ref, d_ref, b_ref, o_ref):
    an = a_ref[...] / (d_ref[...] + 1e-16)
    o_ref[...] = xls_ref[...] * jnp.dot(
        an, b_ref[...], preferred_element_type=jnp.float32, precision=_HIGH
    )


def _film_body(x_ref, c0_ref, c1_ref, g0_ref, g1_ref, o_ref):
    o_ref[...] = (x_ref[...] * c0_ref[...] + c1_ref[...]) * g0_ref[...] + g1_ref[...]


def _gat(h, g, p, srcS, dstS, eaS, amat, bmat):
    xl = h @ p[g + '_Wl'] + p[g + '_bl']
    xr = h @ p[g + '_Wr'] + p[g + '_br']
    ee = eaS @ p[g + '_We']
    xls = xl[srcS]
    xrd = xr[dstS]
    a = _row_pallas(_escore_body, _H, _TILE_E, [xls, xrd, ee], [amat])
    amax = jax.ops.segment_max(a, dstS, num_segments=_N)
    aexp = _row_pallas(_exp_body, _H, _TILE_E, [a, amax[dstS]])
    den = jax.ops.segment_sum(aexp, dstS, num_segments=_N)
    oe = _row_pallas(_wout_body, _H * _C, _TILE_E, [xls, aexp, den[dstS]], [bmat])
    o = jax.ops.segment_sum(oe, dstS, num_segments=_N)
    return o + p[g + '_bias']


_EXPAND = jnp.asarray(np.kron(np.eye(_H), np.ones((1, _C))), dtype=np.float32)
_HEADMASK = np.kron(np.eye(_H), np.ones((_C, 1))).astype(np.float32)


def kernel(x, y, edge_index, edge_weight, t, params):
    p = params
    src0 = edge_index[0]
    dst0 = edge_index[1]

    wt = jnp.where(edge_weight < 0.5, 0.0, edge_weight)
    ewc = jnp.concatenate([wt] * 5, axis=-1)
    ew = jax.nn.relu(ewc @ p['ew1_W'] + p['ew1_b'])
    ew = jax.nn.relu(ew @ p['ew2_W'] + p['ew2_b'])
    ew = ew @ p['ew3_W'] + p['ew3_b']

    freq = jnp.exp(
        -(jnp.log(10000.0) / (_OUT - 1)) * jnp.arange(_OUT, dtype=jnp.float32)
    )
    emb = t * freq[None]
    cond = jnp.concatenate([jnp.sin(emb), jnp.cos(emb)], axis=-1)
    c0 = cond[:, 0::2]
    c1 = cond[:, 1::2]

    h = _bn(x, p['bn0_g'], p['bn0_b'])
    out0 = h @ p['tag_W0']
    ew0 = ew[:, :1]
    hk = h
    for wk in (p['tag_W1'], p['tag_W2']):
        msg = _row_pallas(_scale_body, _D, _TILE_E, [hk[src0], ew0])
        hk = jax.ops.segment_sum(msg, dst0, num_segments=_N)
        out0 = out0 + hk @ wk
    h = out0 + p['tag_b']
    h = jax.nn.relu(h @ p['l0_W'] + p['l0_b'])

    loop = jnp.arange(_N, dtype=src0.dtype)
    srcS = jnp.concatenate([src0, loop])
    dstS = jnp.concatenate([dst0, loop])
    eaS = jnp.concatenate([ew, jnp.ones((_N, _ED), ew.dtype)], axis=0)

    amat1 = p['g1_att'].reshape(-1, 1) * _HEADMASK
    amat2 = p['g2_att'].reshape(-1, 1) * _HEADMASK

    h = _bn(h, p['bn1_g'], p['bn1_b'])
    h = _gat(h, 'g1', p, srcS, dstS, eaS, amat1, _EXPAND)
    h = jax.nn.relu(h @ p['l1_W'] + p['l1_b'])

    h = _bn(h, p['bn2_g'], p['bn2_b'])
    h = _gat(h, 'g2', p, srcS, dstS, eaS, amat2, _EXPAND)

    hg0 = h[:, 0::2]
    hg1 = h[:, 1::2]
    return _row_pallas(_film_body, _D, _TILE_N, [x, c0, c1, hg0, hg1])
